# baseline (speedup 1.0000x reference)
import jax
import jax.numpy as jnp
from jax.experimental import pallas as pl
from jax.experimental.pallas import tpu as pltpu

N = 4096
D = 128
D_OUT = 64
K_IDX = 16609443
HI_BITS = 0x40000000
N_PASSES = 16

BM = 1024
NB = N // BM


def _prep_kernel(x_ref, w_in_ref, xn_ref, h_ref):
    x = x_ref[...]
    nrm = jnp.sqrt(jnp.sum(x * x, axis=1, keepdims=True))
    xn_ref[...] = x / jnp.maximum(nrm, 1e-8)
    h = jax.lax.dot_general(
        x, w_in_ref[...], (((1,), (1,)), ((), ())),
        preferred_element_type=jnp.float32,
    )
    h_ref[...] = jnp.maximum(h, 0.0)


def _sim_kernel(xn_blk_ref, xn_ref, bits_ref):
    s = jax.lax.dot_general(
        xn_blk_ref[...], xn_ref[...], (((1,), (1,)), ((), ())),
        preferred_element_type=jnp.float32,
    )
    bits_ref[...] = pltpu.bitcast(jnp.abs(s), jnp.int32)


def _select_kernel(bits_ref, eps_ref, state_ref, acc_ref):
    p = pl.program_id(0)
    b = pl.program_id(1)
    K1 = float(K_IDX + 1)

    def _thresholds(lo, hi):
        t2 = lo + (hi - lo) // 2
        t1 = lo + (t2 - lo) // 2
        t3 = t2 + (hi - t2) // 2
        return t1, t2, t3

    def _narrow(lo, hi):
        t1, t2, t3 = _thresholds(lo, hi)
        c1 = acc_ref[0]
        c2 = acc_ref[1]
        c3 = acc_ref[2]
        new_hi = jnp.where(c1 >= K1, t1,
                  jnp.where(c2 >= K1, t2,
                   jnp.where(c3 >= K1, t3, hi)))
        new_lo = jnp.where(c1 >= K1, lo,
                  jnp.where(c2 >= K1, t1,
                   jnp.where(c3 >= K1, t2, t3)))
        return new_lo, new_hi

    @pl.when(jnp.logical_and(p == 0, b == 0))
    def _init():
        state_ref[0] = -1
        state_ref[1] = HI_BITS
        acc_ref[0] = 0.0
        acc_ref[1] = 0.0
        acc_ref[2] = 0.0

    @pl.when(jnp.logical_and(p > 0, b == 0))
    def _update():
        new_lo, new_hi = _narrow(state_ref[0], state_ref[1])
        state_ref[0] = new_lo
        state_ref[1] = new_hi
        acc_ref[0] = 0.0
        acc_ref[1] = 0.0
        acc_ref[2] = 0.0

    t1, t2, t3 = _thresholds(state_ref[0], state_ref[1])
    blk = bits_ref[...]
    acc_ref[0] = acc_ref[0] + jnp.sum((blk <= t1).astype(jnp.float32))
    acc_ref[1] = acc_ref[1] + jnp.sum((blk <= t2).astype(jnp.float32))
    acc_ref[2] = acc_ref[2] + jnp.sum((blk <= t3).astype(jnp.float32))

    @pl.when(jnp.logical_and(p == N_PASSES - 1, b == NB - 1))
    def _final():
        _, new_hi = _narrow(state_ref[0], state_ref[1])
        eps_ref[0, 0] = new_hi


def _sage_kernel(eps_ref, bits_ref, h_ref, h_blk_ref, wl_ref, bl_ref,
                 wr_ref, wo_ref, bo_ref, out_ref):
    eps = eps_ref[0]
    mask = (bits_ref[...] >= eps).astype(jnp.float32)
    deg = jnp.sum(mask, axis=1, keepdims=True)
    aggn = jnp.dot(mask, h_ref[...], preferred_element_type=jnp.float32)
    agg = aggn / jnp.maximum(deg, 1.0)
    z = (
        jax.lax.dot_general(
            agg, wl_ref[...], (((1,), (1,)), ((), ())),
            preferred_element_type=jnp.float32,
        )
        + bl_ref[...]
        + jax.lax.dot_general(
            h_blk_ref[...], wr_ref[...], (((1,), (1,)), ((), ())),
            preferred_element_type=jnp.float32,
        )
    )
    h2 = jnp.maximum(z, 0.0)
    o = jax.lax.dot_general(
        h2, wo_ref[...], (((1,), (1,)), ((), ())),
        preferred_element_type=jnp.float32,
    ) + bo_ref[...]
    out_ref[...] = jax.nn.sigmoid(o)


@jax.jit
def kernel(x, W_in, W_l, b_l, W_r, W_out, b_out):
    xn, h = pl.pallas_call(
        _prep_kernel,
        out_shape=(
            jax.ShapeDtypeStruct((N, D), jnp.float32),
            jax.ShapeDtypeStruct((N, D), jnp.float32),
        ),
    )(x, W_in)

    bits = pl.pallas_call(
        _sim_kernel,
        grid=(NB,),
        in_specs=[
            pl.BlockSpec((BM, D), lambda i: (i, 0)),
            pl.BlockSpec((N, D), lambda i: (0, 0)),
        ],
        out_specs=pl.BlockSpec((BM, N), lambda i: (i, 0)),
        out_shape=jax.ShapeDtypeStruct((N, N), jnp.int32),
    )(xn, xn)

    eps_bits = pl.pallas_call(
        _select_kernel,
        grid=(N_PASSES, NB),
        in_specs=[pl.BlockSpec((BM, N), lambda p, b: (b, 0))],
        out_specs=pl.BlockSpec(memory_space=pltpu.SMEM),
        out_shape=jax.ShapeDtypeStruct((1, 1), jnp.int32),
        scratch_shapes=[
            pltpu.SMEM((2,), jnp.int32),
            pltpu.SMEM((3,), jnp.float32),
        ],
    )(bits)

    out = pl.pallas_call(
        _sage_kernel,
        grid=(NB,),
        in_specs=[
            pl.BlockSpec(memory_space=pltpu.SMEM),
            pl.BlockSpec((BM, N), lambda i: (i, 0)),
            pl.BlockSpec((N, D), lambda i: (0, 0)),
            pl.BlockSpec((BM, D), lambda i: (i, 0)),
            pl.BlockSpec((D, D), lambda i: (0, 0)),
            pl.BlockSpec((1, D), lambda i: (0, 0)),
            pl.BlockSpec((D, D), lambda i: (0, 0)),
            pl.BlockSpec((D_OUT, D), lambda i: (0, 0)),
            pl.BlockSpec((1, D_OUT), lambda i: (0, 0)),
        ],
        out_specs=pl.BlockSpec((BM, D_OUT), lambda i: (i, 0)),
        out_shape=jax.ShapeDtypeStruct((N, D_OUT), jnp.float32),
    )(
        eps_bits.reshape(-1), bits, h, h,
        W_l, b_l.reshape(1, D), W_r, W_out, b_out.reshape(1, D_OUT),
    )
    return out
